# baseline (speedup 1.0000x reference)
import functools

import jax
import jax.numpy as jnp
from jax import lax
from jax.experimental import pallas as pl
from jax.experimental.pallas import tpu as pltpu
from jax.experimental.pallas import tpu_sc as plsc

B = 16384
NHALF = 4
BH = B // NHALF
NW = 32
ROWS_PER_W = BH // NW
CHUNK = 128
DG = 32
DM = 128

_HI = -65536


def _pack_ops(t):
    h = t.shape[1] // 2
    lo = lax.bitcast_convert_type(t[:, :h].astype(jnp.bfloat16), jnp.uint16)
    hi = lax.bitcast_convert_type(t[:, h:].astype(jnp.bfloat16), jnp.uint16)
    return lax.bitcast_convert_type(
        lo.astype(jnp.uint32) | (hi.astype(jnp.uint32) << 16), jnp.int32)


def _pack_body(eu, ei, eug, eig, eu_out, ei_out, eug_out, eig_out):
    eu_out[...] = _pack_ops(eu[...])
    ei_out[...] = _pack_ops(ei[...])
    zu = jnp.zeros((eug.shape[0], DM - DG), jnp.int32)
    zi = jnp.zeros((eig.shape[0], DM - DG), jnp.int32)
    eug_out[...] = jnp.concatenate([_pack_ops(eug[...]), zu], axis=1)
    eig_out[...] = jnp.concatenate([_pack_ops(eig[...]), zi], axis=1)


def _pack_call(eu_mlp, ei_mlp, eu_gmf, ei_gmf):
    return pl.pallas_call(
        _pack_body,
        out_shape=[
            jax.ShapeDtypeStruct((eu_mlp.shape[0], DM), jnp.int32),
            jax.ShapeDtypeStruct((ei_mlp.shape[0], DM), jnp.int32),
            jax.ShapeDtypeStruct((eu_gmf.shape[0], DM), jnp.int32),
            jax.ShapeDtypeStruct((ei_gmf.shape[0], DM), jnp.int32),
        ],
    )(eu_mlp, ei_mlp, eu_gmf, ei_gmf)


def _sc_gather_body(uidx, sidx, eug, eum, eig, eim,
                    ug_out, um_out, ig_out, im_out,
                    idx_u, idx_s, r_ug, r_um, r_ig, r_im, sem):
    wid = lax.axis_index("s") * 2 + lax.axis_index("c")
    base = wid * ROWS_PER_W
    for k in range(ROWS_PER_W // CHUNK):
        off = base + k * CHUNK
        pltpu.sync_copy(uidx.at[pl.ds(off, CHUNK)], idx_u)
        pltpu.sync_copy(sidx.at[pl.ds(off, CHUNK)], idx_s)
        h1 = pltpu.async_copy(eug.at[idx_u], r_ug, sem)
        h2 = pltpu.async_copy(eum.at[idx_u], r_um, sem)
        h3 = pltpu.async_copy(eig.at[idx_s], r_ig, sem)
        h4 = pltpu.async_copy(eim.at[idx_s], r_im, sem)
        h1.wait()
        h2.wait()
        h3.wait()
        h4.wait()
        pltpu.sync_copy(r_ug, ug_out.at[pl.ds(off, CHUNK)])
        pltpu.sync_copy(r_um, um_out.at[pl.ds(off, CHUNK)])
        pltpu.sync_copy(r_ig, ig_out.at[pl.ds(off, CHUNK)])
        pltpu.sync_copy(r_im, im_out.at[pl.ds(off, CHUNK)])


_sc_gather = pl.kernel(
    _sc_gather_body,
    mesh=plsc.VectorSubcoreMesh(core_axis_name="c", subcore_axis_name="s"),
    out_type=[
        jax.ShapeDtypeStruct((BH, DM), jnp.int32),
        jax.ShapeDtypeStruct((BH, DM), jnp.int32),
        jax.ShapeDtypeStruct((BH, DM), jnp.int32),
        jax.ShapeDtypeStruct((BH, DM), jnp.int32),
    ],
    scratch_types=[
        pltpu.VMEM((CHUNK,), jnp.int32),
        pltpu.VMEM((CHUNK,), jnp.int32),
        pltpu.VMEM((CHUNK, DM), jnp.int32),
        pltpu.VMEM((CHUNK, DM), jnp.int32),
        pltpu.VMEM((CHUNK, DM), jnp.int32),
        pltpu.VMEM((CHUNK, DM), jnp.int32),
        pltpu.SemaphoreType.DMA,
    ],
    compiler_params=pltpu.CompilerParams(use_tc_tiling_on_sc=True),
)


BBLK = 2048


def _unpack(p):
    even = lax.bitcast_convert_type(p << 16, jnp.float32)
    odd = lax.bitcast_convert_type(p & _HI, jnp.float32)
    return even, odd


def _tc_body(ug, um, ig, im, w1ae, w1ao, w1be, w1bo, b1, w2, b2, w3, b3,
             wpae, wpao, wpb, bp, out):
    f32 = jnp.float32
    ume, umo = _unpack(um[...])
    ime, imo = _unpack(im[...])
    h = jnp.dot(ume, w1ae[...], preferred_element_type=f32)
    h += jnp.dot(umo, w1ao[...], preferred_element_type=f32)
    h += jnp.dot(ime, w1be[...], preferred_element_type=f32)
    h += jnp.dot(imo, w1bo[...], preferred_element_type=f32)
    h = jnp.maximum(h + b1[...], 0.0)
    h = jnp.maximum(jnp.dot(h, w2[...], preferred_element_type=f32) + b2[...], 0.0)
    h3 = jnp.maximum(jnp.dot(h, w3[...], preferred_element_type=f32) + b3[...], 0.0)
    uge, ugo = _unpack(ug[:, :DG])
    ige, igo = _unpack(ig[:, :DG])
    pred = (jnp.sum(uge * ige * wpae[...], axis=-1, keepdims=True)
            + jnp.sum(ugo * igo * wpao[...], axis=-1, keepdims=True)
            + jnp.sum(h3 * wpb[...], axis=-1, keepdims=True)
            + bp[0, 0])
    out[...] = pred


def _tc_call(ug, um, ig, im, w1ae, w1ao, w1be, w1bo, b1, w2, b2, w3, b3,
             wpae, wpao, wpb, bp):
    nblk = BH // BBLK
    row = lambda i: (i, 0)
    rep = lambda i: (0, 0)
    return pl.pallas_call(
        _tc_body,
        grid=(nblk,),
        in_specs=[
            pl.BlockSpec((BBLK, DM), row),
            pl.BlockSpec((BBLK, DM), row),
            pl.BlockSpec((BBLK, DM), row),
            pl.BlockSpec((BBLK, DM), row),
            pl.BlockSpec((128, 256), rep),
            pl.BlockSpec((128, 256), rep),
            pl.BlockSpec((128, 256), rep),
            pl.BlockSpec((128, 256), rep),
            pl.BlockSpec((1, 256), rep),
            pl.BlockSpec((256, 128), rep),
            pl.BlockSpec((1, 128), rep),
            pl.BlockSpec((128, 64), rep),
            pl.BlockSpec((1, 64), rep),
            pl.BlockSpec((1, 32), rep),
            pl.BlockSpec((1, 32), rep),
            pl.BlockSpec((1, 64), rep),
            pl.BlockSpec((1, 1), rep),
        ],
        out_specs=pl.BlockSpec((BBLK, 1), row),
        out_shape=jax.ShapeDtypeStruct((BH, 1), jnp.float32),
        compiler_params=pltpu.CompilerParams(
            dimension_semantics=("parallel",)),
    )(ug, um, ig, im, w1ae, w1ao, w1be, w1bo, b1, w2, b2, w3, b3,
      wpae, wpao, wpb, bp)


def kernel(userIdx, servIdx, eu_gmf, eu_mlp, ei_gmf, ei_mlp,
           W1, b1, W2, b2, W3, b3, Wp, bp):
    uidx = userIdx.astype(jnp.int32)
    sidx = servIdx.astype(jnp.int32)

    eum_p, eim_p, eug_p, eig_p = _pack_call(eu_mlp, ei_mlp, eu_gmf, ei_gmf)
    w1a, w1b = W1[:256], W1[256:]
    wp = Wp[:, 0]
    weights = (w1a[:128], w1a[128:], w1b[:128], w1b[128:],
               b1.reshape(1, 256), W2, b2.reshape(1, 128), W3,
               b3.reshape(1, 64),
               wp[0:32].reshape(1, 32), wp[32:64].reshape(1, 32),
               wp[64:128].reshape(1, 64),
               bp.reshape(1, 1))
    outs = []
    for k in range(NHALF):
        ug, um, ig, im = _sc_gather(uidx[k * BH:(k + 1) * BH],
                                    sidx[k * BH:(k + 1) * BH],
                                    eug_p, eum_p, eig_p, eim_p)
        outs.append(_tc_call(ug, um, ig, im, *weights))
    return jnp.concatenate(outs, axis=0).reshape(-1)


# speedup vs baseline: 1.0178x; 1.0178x over previous
import functools

import jax
import jax.numpy as jnp
from jax import lax
from jax.experimental import pallas as pl
from jax.experimental.pallas import tpu as pltpu
from jax.experimental.pallas import tpu_sc as plsc

B = 16384
NHALF = 2
BH = B // NHALF
NW = 32
ROWS_PER_W = BH // NW
CHUNK = 128
DG = 32
DM = 128

_HI = -65536


def _pack_ops(t):
    h = t.shape[1] // 2
    lo = lax.bitcast_convert_type(t[:, :h].astype(jnp.bfloat16), jnp.uint16)
    hi = lax.bitcast_convert_type(t[:, h:].astype(jnp.bfloat16), jnp.uint16)
    return lax.bitcast_convert_type(
        lo.astype(jnp.uint32) | (hi.astype(jnp.uint32) << 16), jnp.int32)


def _pack_body(eu, ei, eug, eig, eu_out, ei_out, eug_out, eig_out):
    eu_out[...] = _pack_ops(eu[...])
    ei_out[...] = _pack_ops(ei[...])
    zu = jnp.zeros((eug.shape[0], DM - DG), jnp.int32)
    zi = jnp.zeros((eig.shape[0], DM - DG), jnp.int32)
    eug_out[...] = jnp.concatenate([_pack_ops(eug[...]), zu], axis=1)
    eig_out[...] = jnp.concatenate([_pack_ops(eig[...]), zi], axis=1)


def _pack_call(eu_mlp, ei_mlp, eu_gmf, ei_gmf):
    return pl.pallas_call(
        _pack_body,
        out_shape=[
            jax.ShapeDtypeStruct((eu_mlp.shape[0], DM), jnp.int32),
            jax.ShapeDtypeStruct((ei_mlp.shape[0], DM), jnp.int32),
            jax.ShapeDtypeStruct((eu_gmf.shape[0], DM), jnp.int32),
            jax.ShapeDtypeStruct((ei_gmf.shape[0], DM), jnp.int32),
        ],
    )(eu_mlp, ei_mlp, eu_gmf, ei_gmf)


def _sc_gather_body(uidx, sidx, eug, eum, eig, eim,
                    ug_out, um_out, ig_out, im_out,
                    idx_u, idx_s, r_ug, r_um, r_ig, r_im, sem):
    wid = lax.axis_index("s") * 2 + lax.axis_index("c")
    base = wid * ROWS_PER_W
    for k in range(ROWS_PER_W // CHUNK):
        off = base + k * CHUNK
        pltpu.sync_copy(uidx.at[pl.ds(off, CHUNK)], idx_u)
        pltpu.sync_copy(sidx.at[pl.ds(off, CHUNK)], idx_s)
        h1 = pltpu.async_copy(eug.at[idx_u], r_ug, sem)
        h2 = pltpu.async_copy(eum.at[idx_u], r_um, sem)
        h3 = pltpu.async_copy(eig.at[idx_s], r_ig, sem)
        h4 = pltpu.async_copy(eim.at[idx_s], r_im, sem)
        h1.wait()
        h2.wait()
        h3.wait()
        h4.wait()
        pltpu.sync_copy(r_ug, ug_out.at[pl.ds(off, CHUNK)])
        pltpu.sync_copy(r_um, um_out.at[pl.ds(off, CHUNK)])
        pltpu.sync_copy(r_ig, ig_out.at[pl.ds(off, CHUNK)])
        pltpu.sync_copy(r_im, im_out.at[pl.ds(off, CHUNK)])


_sc_gather = pl.kernel(
    _sc_gather_body,
    mesh=plsc.VectorSubcoreMesh(core_axis_name="c", subcore_axis_name="s"),
    out_type=[
        jax.ShapeDtypeStruct((BH, DM), jnp.int32),
        jax.ShapeDtypeStruct((BH, DM), jnp.int32),
        jax.ShapeDtypeStruct((BH, DM), jnp.int32),
        jax.ShapeDtypeStruct((BH, DM), jnp.int32),
    ],
    scratch_types=[
        pltpu.VMEM((CHUNK,), jnp.int32),
        pltpu.VMEM((CHUNK,), jnp.int32),
        pltpu.VMEM((CHUNK, DM), jnp.int32),
        pltpu.VMEM((CHUNK, DM), jnp.int32),
        pltpu.VMEM((CHUNK, DM), jnp.int32),
        pltpu.VMEM((CHUNK, DM), jnp.int32),
        pltpu.SemaphoreType.DMA,
    ],
    compiler_params=pltpu.CompilerParams(use_tc_tiling_on_sc=True),
)


BBLK = 2048


def _unpack(p):
    even = lax.bitcast_convert_type(p << 16, jnp.float32)
    odd = lax.bitcast_convert_type(p & _HI, jnp.float32)
    return even, odd


def _tc_body(ug, um, ig, im, w1ae, w1ao, w1be, w1bo, b1, w2, b2, w3, b3,
             wpae, wpao, wpb, bp, out):
    f32 = jnp.float32
    ume, umo = _unpack(um[...])
    ime, imo = _unpack(im[...])
    h = jnp.dot(ume, w1ae[...], preferred_element_type=f32)
    h += jnp.dot(umo, w1ao[...], preferred_element_type=f32)
    h += jnp.dot(ime, w1be[...], preferred_element_type=f32)
    h += jnp.dot(imo, w1bo[...], preferred_element_type=f32)
    h = jnp.maximum(h + b1[...], 0.0)
    h = jnp.maximum(jnp.dot(h, w2[...], preferred_element_type=f32) + b2[...], 0.0)
    h3 = jnp.maximum(jnp.dot(h, w3[...], preferred_element_type=f32) + b3[...], 0.0)
    uge, ugo = _unpack(ug[:, :DG])
    ige, igo = _unpack(ig[:, :DG])
    pred = (jnp.sum(uge * ige * wpae[...], axis=-1, keepdims=True)
            + jnp.sum(ugo * igo * wpao[...], axis=-1, keepdims=True)
            + jnp.sum(h3 * wpb[...], axis=-1, keepdims=True)
            + bp[0, 0])
    out[...] = pred


def _tc_call(ug, um, ig, im, w1ae, w1ao, w1be, w1bo, b1, w2, b2, w3, b3,
             wpae, wpao, wpb, bp):
    nblk = BH // BBLK
    row = lambda i: (i, 0)
    rep = lambda i: (0, 0)
    return pl.pallas_call(
        _tc_body,
        grid=(nblk,),
        in_specs=[
            pl.BlockSpec((BBLK, DM), row),
            pl.BlockSpec((BBLK, DM), row),
            pl.BlockSpec((BBLK, DM), row),
            pl.BlockSpec((BBLK, DM), row),
            pl.BlockSpec((128, 256), rep),
            pl.BlockSpec((128, 256), rep),
            pl.BlockSpec((128, 256), rep),
            pl.BlockSpec((128, 256), rep),
            pl.BlockSpec((1, 256), rep),
            pl.BlockSpec((256, 128), rep),
            pl.BlockSpec((1, 128), rep),
            pl.BlockSpec((128, 64), rep),
            pl.BlockSpec((1, 64), rep),
            pl.BlockSpec((1, 32), rep),
            pl.BlockSpec((1, 32), rep),
            pl.BlockSpec((1, 64), rep),
            pl.BlockSpec((1, 1), rep),
        ],
        out_specs=pl.BlockSpec((BBLK, 1), row),
        out_shape=jax.ShapeDtypeStruct((BH, 1), jnp.float32),
        compiler_params=pltpu.CompilerParams(
            dimension_semantics=("parallel",)),
    )(ug, um, ig, im, w1ae, w1ao, w1be, w1bo, b1, w2, b2, w3, b3,
      wpae, wpao, wpb, bp)


def kernel(userIdx, servIdx, eu_gmf, eu_mlp, ei_gmf, ei_mlp,
           W1, b1, W2, b2, W3, b3, Wp, bp):
    uidx = userIdx.astype(jnp.int32)
    sidx = servIdx.astype(jnp.int32)

    eum_p, eim_p, eug_p, eig_p = _pack_call(eu_mlp, ei_mlp, eu_gmf, ei_gmf)
    w1a, w1b = W1[:256], W1[256:]
    wp = Wp[:, 0]
    weights = (w1a[:128], w1a[128:], w1b[:128], w1b[128:],
               b1.reshape(1, 256), W2, b2.reshape(1, 128), W3,
               b3.reshape(1, 64),
               wp[0:32].reshape(1, 32), wp[32:64].reshape(1, 32),
               wp[64:128].reshape(1, 64),
               bp.reshape(1, 1))
    outs = []
    for k in range(NHALF):
        ug, um, ig, im = _sc_gather(uidx[k * BH:(k + 1) * BH],
                                    sidx[k * BH:(k + 1) * BH],
                                    eug_p, eum_p, eig_p, eim_p)
        outs.append(_tc_call(ug, um, ig, im, *weights))
    return jnp.concatenate(outs, axis=0).reshape(-1)


# speedup vs baseline: 1.0443x; 1.0260x over previous
import functools

import jax
import jax.numpy as jnp
from jax import lax
from jax.experimental import pallas as pl
from jax.experimental.pallas import tpu as pltpu
from jax.experimental.pallas import tpu_sc as plsc

B = 16384
NHALF = 2
BH = B // NHALF
NW = 32
ROWS_PER_W = BH // NW
CHUNK = 128
DG = 32
DM = 128

_HI = -65536


def _pack_ops(t):
    h = t.shape[1] // 2
    lo = lax.bitcast_convert_type(t[:, :h].astype(jnp.bfloat16), jnp.uint16)
    hi = lax.bitcast_convert_type(t[:, h:].astype(jnp.bfloat16), jnp.uint16)
    return lax.bitcast_convert_type(
        lo.astype(jnp.uint32) | (hi.astype(jnp.uint32) << 16), jnp.int32)


def _pack_body(eu, ei, eu_out, ei_out):
    eu_out[...] = _pack_ops(eu[...])
    ei_out[...] = _pack_ops(ei[...])


def _pack_call(eu_mlp, ei_mlp):
    return pl.pallas_call(
        _pack_body,
        out_shape=[
            jax.ShapeDtypeStruct((eu_mlp.shape[0], DM), jnp.int32),
            jax.ShapeDtypeStruct((ei_mlp.shape[0], DM), jnp.int32),
        ],
    )(eu_mlp, ei_mlp)


def _sc_gather_body(uidx, sidx, eug, eum, eig, eim,
                    ug_out, um_out, ig_out, im_out,
                    idx_u, idx_s, r_ug, r_um, r_ig, r_im, sem):
    wid = lax.axis_index("s") * 2 + lax.axis_index("c")
    base = wid * ROWS_PER_W
    for k in range(ROWS_PER_W // CHUNK):
        off = base + k * CHUNK
        pltpu.sync_copy(uidx.at[pl.ds(off, CHUNK)], idx_u)
        pltpu.sync_copy(sidx.at[pl.ds(off, CHUNK)], idx_s)
        h1 = pltpu.async_copy(eug.at[idx_u], r_ug, sem)
        h2 = pltpu.async_copy(eum.at[idx_u], r_um, sem)
        h3 = pltpu.async_copy(eig.at[idx_s], r_ig, sem)
        h4 = pltpu.async_copy(eim.at[idx_s], r_im, sem)
        h1.wait()
        h2.wait()
        h3.wait()
        h4.wait()
        pltpu.sync_copy(r_ug, ug_out.at[pl.ds(off, CHUNK)])
        pltpu.sync_copy(r_um, um_out.at[pl.ds(off, CHUNK)])
        pltpu.sync_copy(r_ig, ig_out.at[pl.ds(off, CHUNK)])
        pltpu.sync_copy(r_im, im_out.at[pl.ds(off, CHUNK)])


_sc_gather = pl.kernel(
    _sc_gather_body,
    mesh=plsc.VectorSubcoreMesh(core_axis_name="c", subcore_axis_name="s"),
    out_type=[
        jax.ShapeDtypeStruct((BH, DM), jnp.int32),
        jax.ShapeDtypeStruct((BH, DM), jnp.int32),
        jax.ShapeDtypeStruct((BH, DM), jnp.int32),
        jax.ShapeDtypeStruct((BH, DM), jnp.int32),
    ],
    scratch_types=[
        pltpu.VMEM((CHUNK,), jnp.int32),
        pltpu.VMEM((CHUNK,), jnp.int32),
        pltpu.VMEM((CHUNK, DM), jnp.int32),
        pltpu.VMEM((CHUNK, DM), jnp.int32),
        pltpu.VMEM((CHUNK, DM), jnp.int32),
        pltpu.VMEM((CHUNK, DM), jnp.int32),
        pltpu.SemaphoreType.DMA,
    ],
    compiler_params=pltpu.CompilerParams(use_tc_tiling_on_sc=True),
)


BBLK = 2048


def _unpack(p):
    even = lax.bitcast_convert_type(p << 16, jnp.float32)
    odd = lax.bitcast_convert_type(p & _HI, jnp.float32)
    return even, odd


def _tc_body(ug, um, ig, im, w1ae, w1ao, w1be, w1bo, b1, w2, b2, w3, b3,
             wpae, wpao, wpb, bp, out):
    f32 = jnp.float32
    ume, umo = _unpack(um[...])
    ime, imo = _unpack(im[...])
    h = jnp.dot(ume, w1ae[...], preferred_element_type=f32)
    h += jnp.dot(umo, w1ao[...], preferred_element_type=f32)
    h += jnp.dot(ime, w1be[...], preferred_element_type=f32)
    h += jnp.dot(imo, w1bo[...], preferred_element_type=f32)
    h = jnp.maximum(h + b1[...], 0.0)
    h = jnp.maximum(jnp.dot(h, w2[...], preferred_element_type=f32) + b2[...], 0.0)
    h3 = jnp.maximum(jnp.dot(h, w3[...], preferred_element_type=f32) + b3[...], 0.0)
    uge, ugo = _unpack(ug[:, :DG])
    ige, igo = _unpack(ig[:, :DG])
    pred = (jnp.sum(uge * ige * wpae[...], axis=-1, keepdims=True)
            + jnp.sum(ugo * igo * wpao[...], axis=-1, keepdims=True)
            + jnp.sum(h3 * wpb[...], axis=-1, keepdims=True)
            + bp[0, 0])
    out[...] = pred


def _tc_call(ug, um, ig, im, w1ae, w1ao, w1be, w1bo, b1, w2, b2, w3, b3,
             wpae, wpao, wpb, bp):
    nblk = BH // BBLK
    row = lambda i: (i, 0)
    rep = lambda i: (0, 0)
    return pl.pallas_call(
        _tc_body,
        grid=(nblk,),
        in_specs=[
            pl.BlockSpec((BBLK, DM), row),
            pl.BlockSpec((BBLK, DM), row),
            pl.BlockSpec((BBLK, DM), row),
            pl.BlockSpec((BBLK, DM), row),
            pl.BlockSpec((128, 256), rep),
            pl.BlockSpec((128, 256), rep),
            pl.BlockSpec((128, 256), rep),
            pl.BlockSpec((128, 256), rep),
            pl.BlockSpec((1, 256), rep),
            pl.BlockSpec((256, 128), rep),
            pl.BlockSpec((1, 128), rep),
            pl.BlockSpec((128, 64), rep),
            pl.BlockSpec((1, 64), rep),
            pl.BlockSpec((1, 32), rep),
            pl.BlockSpec((1, 32), rep),
            pl.BlockSpec((1, 64), rep),
            pl.BlockSpec((1, 1), rep),
        ],
        out_specs=pl.BlockSpec((BBLK, 1), row),
        out_shape=jax.ShapeDtypeStruct((BH, 1), jnp.float32),
        compiler_params=pltpu.CompilerParams(
            dimension_semantics=("parallel",)),
    )(ug, um, ig, im, w1ae, w1ao, w1be, w1bo, b1, w2, b2, w3, b3,
      wpae, wpao, wpb, bp)


def kernel(userIdx, servIdx, eu_gmf, eu_mlp, ei_gmf, ei_mlp,
           W1, b1, W2, b2, W3, b3, Wp, bp):
    uidx = userIdx.astype(jnp.int32)
    sidx = servIdx.astype(jnp.int32)

    def pad128(t):
        return jnp.pad(t, ((0, 0), (0, DM - t.shape[1])))

    eum_p, eim_p = _pack_call(eu_mlp, ei_mlp)
    eug_p = pad128(_pack_ops(eu_gmf))
    eig_p = pad128(_pack_ops(ei_gmf))
    w1a, w1b = W1[:256], W1[256:]
    wp = Wp[:, 0]
    weights = (w1a[:128], w1a[128:], w1b[:128], w1b[128:],
               b1.reshape(1, 256), W2, b2.reshape(1, 128), W3,
               b3.reshape(1, 64),
               wp[0:32].reshape(1, 32), wp[32:64].reshape(1, 32),
               wp[64:128].reshape(1, 64),
               bp.reshape(1, 1))
    outs = []
    for k in range(NHALF):
        ug, um, ig, im = _sc_gather(uidx[k * BH:(k + 1) * BH],
                                    sidx[k * BH:(k + 1) * BH],
                                    eug_p, eum_p, eig_p, eim_p)
        outs.append(_tc_call(ug, um, ig, im, *weights))
    return jnp.concatenate(outs, axis=0).reshape(-1)


# speedup vs baseline: 1.0471x; 1.0027x over previous
import jax
import jax.numpy as jnp
from jax import lax
from jax.experimental import pallas as pl
from jax.experimental.pallas import tpu as pltpu
from jax.experimental.pallas import tpu_sc as plsc

B = 16384
NHALF = 2
BH = B // NHALF
NW = 32
ROWS_PER_W = BH // NW
CHUNK = 128
DG = 32
DM = 128

_HI = -65536


def _pack_ops(t):
    h = t.shape[1] // 2
    lo = lax.bitcast_convert_type(t[:, :h].astype(jnp.bfloat16), jnp.uint16)
    hi = lax.bitcast_convert_type(t[:, h:].astype(jnp.bfloat16), jnp.uint16)
    return lax.bitcast_convert_type(
        lo.astype(jnp.uint32) | (hi.astype(jnp.uint32) << 16), jnp.int32)


def _pack_body(eu, ei, eu_out, ei_out):
    eu_out[...] = _pack_ops(eu[...])
    ei_out[...] = _pack_ops(ei[...])


def _pack_call(eu_mlp, ei_mlp):
    return pl.pallas_call(
        _pack_body,
        out_shape=[
            jax.ShapeDtypeStruct((eu_mlp.shape[0], DM), jnp.int32),
            jax.ShapeDtypeStruct((ei_mlp.shape[0], DM), jnp.int32),
        ],
    )(eu_mlp, ei_mlp)


def _sc_gather_body(uidx, sidx, eug, eum, eig, eim,
                    ug_out, um_out, ig_out, im_out,
                    idx_u, idx_s, r_ug, r_um, r_ig, r_im, sem):
    wid = lax.axis_index("s") * 2 + lax.axis_index("c")
    base = wid * ROWS_PER_W
    for k in range(ROWS_PER_W // CHUNK):
        off = base + k * CHUNK
        pltpu.sync_copy(uidx.at[pl.ds(off, CHUNK)], idx_u)
        pltpu.sync_copy(sidx.at[pl.ds(off, CHUNK)], idx_s)
        h1 = pltpu.async_copy(eug.at[idx_u], r_ug, sem)
        h2 = pltpu.async_copy(eum.at[idx_u], r_um, sem)
        h3 = pltpu.async_copy(eig.at[idx_s], r_ig, sem)
        h4 = pltpu.async_copy(eim.at[idx_s], r_im, sem)
        h1.wait()
        h2.wait()
        h3.wait()
        h4.wait()
        pltpu.sync_copy(r_ug, ug_out.at[pl.ds(off, CHUNK)])
        pltpu.sync_copy(r_um, um_out.at[pl.ds(off, CHUNK)])
        pltpu.sync_copy(r_ig, ig_out.at[pl.ds(off, CHUNK)])
        pltpu.sync_copy(r_im, im_out.at[pl.ds(off, CHUNK)])


_sc_gather = pl.kernel(
    _sc_gather_body,
    mesh=plsc.VectorSubcoreMesh(core_axis_name="c", subcore_axis_name="s"),
    out_type=[
        jax.ShapeDtypeStruct((BH, DM), jnp.int32),
        jax.ShapeDtypeStruct((BH, DM), jnp.int32),
        jax.ShapeDtypeStruct((BH, DM), jnp.int32),
        jax.ShapeDtypeStruct((BH, DM), jnp.int32),
    ],
    scratch_types=[
        pltpu.VMEM((CHUNK,), jnp.int32),
        pltpu.VMEM((CHUNK,), jnp.int32),
        pltpu.VMEM((CHUNK, DM), jnp.int32),
        pltpu.VMEM((CHUNK, DM), jnp.int32),
        pltpu.VMEM((CHUNK, DM), jnp.int32),
        pltpu.VMEM((CHUNK, DM), jnp.int32),
        pltpu.SemaphoreType.DMA,
    ],
    compiler_params=pltpu.CompilerParams(use_tc_tiling_on_sc=True),
)


BBLK = 2048


def _unpack(p):
    even = lax.bitcast_convert_type(p << 16, jnp.float32)
    odd = lax.bitcast_convert_type(p & _HI, jnp.float32)
    return even, odd


def _tc_body(ug, um, ig, im, w1ae, w1ao, w1be, w1bo, b1, w2, b2, w3, b3,
             wpae, wpao, wpb, bp, out):
    f32 = jnp.float32
    ume, umo = _unpack(um[...])
    ime, imo = _unpack(im[...])
    h = jnp.dot(ume, w1ae[...], preferred_element_type=f32)
    h += jnp.dot(umo, w1ao[...], preferred_element_type=f32)
    h += jnp.dot(ime, w1be[...], preferred_element_type=f32)
    h += jnp.dot(imo, w1bo[...], preferred_element_type=f32)
    h = jnp.maximum(h + b1[...], 0.0)
    h = jnp.maximum(jnp.dot(h, w2[...], preferred_element_type=f32) + b2[...], 0.0)
    h3 = jnp.maximum(jnp.dot(h, w3[...], preferred_element_type=f32) + b3[...], 0.0)
    uge, ugo = _unpack(ug[:, :DG])
    ige, igo = _unpack(ig[:, :DG])
    pred = (jnp.sum(uge * ige * wpae[...], axis=-1, keepdims=True)
            + jnp.sum(ugo * igo * wpao[...], axis=-1, keepdims=True)
            + jnp.sum(h3 * wpb[...], axis=-1, keepdims=True)
            + bp[0, 0])
    out[...] = pred


def _tc_call(ug, um, ig, im, w1ae, w1ao, w1be, w1bo, b1, w2, b2, w3, b3,
             wpae, wpao, wpb, bp):
    nblk = BH // BBLK
    row = lambda i: (i, 0)
    rep = lambda i: (0, 0)
    return pl.pallas_call(
        _tc_body,
        grid=(nblk,),
        in_specs=[
            pl.BlockSpec((BBLK, DM), row),
            pl.BlockSpec((BBLK, DM), row),
            pl.BlockSpec((BBLK, DM), row),
            pl.BlockSpec((BBLK, DM), row),
            pl.BlockSpec((128, 256), rep),
            pl.BlockSpec((128, 256), rep),
            pl.BlockSpec((128, 256), rep),
            pl.BlockSpec((128, 256), rep),
            pl.BlockSpec((1, 256), rep),
            pl.BlockSpec((256, 128), rep),
            pl.BlockSpec((1, 128), rep),
            pl.BlockSpec((128, 64), rep),
            pl.BlockSpec((1, 64), rep),
            pl.BlockSpec((1, 32), rep),
            pl.BlockSpec((1, 32), rep),
            pl.BlockSpec((1, 64), rep),
            pl.BlockSpec((1, 1), rep),
        ],
        out_specs=pl.BlockSpec((BBLK, 1), row),
        out_shape=jax.ShapeDtypeStruct((BH, 1), jnp.float32),
        compiler_params=pltpu.CompilerParams(
            dimension_semantics=("parallel",)),
    )(ug, um, ig, im, w1ae, w1ao, w1be, w1bo, b1, w2, b2, w3, b3,
      wpae, wpao, wpb, bp)


def kernel(userIdx, servIdx, eu_gmf, eu_mlp, ei_gmf, ei_mlp,
           W1, b1, W2, b2, W3, b3, Wp, bp):
    uidx = userIdx.astype(jnp.int32)
    sidx = servIdx.astype(jnp.int32)

    def pad128(t):
        return jnp.pad(t, ((0, 0), (0, DM - t.shape[1])))

    eum_p, eim_p = _pack_call(eu_mlp, ei_mlp)
    eug_p = pad128(_pack_ops(eu_gmf))
    eig_p = pad128(_pack_ops(ei_gmf))
    w1a, w1b = W1[:256], W1[256:]
    wp = Wp[:, 0]
    weights = (w1a[:128], w1a[128:], w1b[:128], w1b[128:],
               b1.reshape(1, 256), W2, b2.reshape(1, 128), W3,
               b3.reshape(1, 64),
               wp[0:32].reshape(1, 32), wp[32:64].reshape(1, 32),
               wp[64:128].reshape(1, 64),
               bp.reshape(1, 1))
    outs = []
    for k in range(NHALF):
        ug, um, ig, im = _sc_gather(uidx[k * BH:(k + 1) * BH],
                                    sidx[k * BH:(k + 1) * BH],
                                    eug_p, eum_p, eig_p, eim_p)
        outs.append(_tc_call(ug, um, ig, im, *weights))
    return jnp.concatenate(outs, axis=0).reshape(-1)
